# baseline (speedup 1.0000x reference)
import functools

import jax
import jax.numpy as jnp
from jax import lax
from jax.experimental import pallas as pl
from jax.experimental.pallas import tpu as pltpu
from jax.experimental.pallas import tpu_sc as plsc

N_NODES = 50000
N_EDGES = 800000
RBF_DIM = 128
DIM = 64
BETA = 0.5
THRESHOLD = 14.0


_MLP_BLOCK = 8000


def _mlp_body(rbf_ref, w1_ref, b1_ref, w2_ref, b2_ref, h_ref):
    x = rbf_ref[...]
    h1 = jnp.dot(x, w1_ref[...], preferred_element_type=jnp.float32) + b1_ref[...]
    bx = BETA * h1
    sp = (jnp.maximum(bx, 0.0) + jnp.log1p(jnp.exp(-jnp.abs(bx)))) / BETA
    act = jnp.where(bx > THRESHOLD, h1, sp)
    h_ref[...] = (
        jnp.dot(act, w2_ref[...], preferred_element_type=jnp.float32) + b2_ref[...]
    )


def _edge_mlp(rbf, W1, b1, W2, b2):
    nblk = N_EDGES // _MLP_BLOCK
    return pl.pallas_call(
        _mlp_body,
        grid=(nblk,),
        in_specs=[
            pl.BlockSpec((_MLP_BLOCK, RBF_DIM), lambda i: (i, 0)),
            pl.BlockSpec((RBF_DIM, DIM), lambda i: (0, 0)),
            pl.BlockSpec((1, DIM), lambda i: (0, 0)),
            pl.BlockSpec((DIM, DIM), lambda i: (0, 0)),
            pl.BlockSpec((1, DIM), lambda i: (0, 0)),
        ],
        out_specs=pl.BlockSpec((_MLP_BLOCK, DIM), lambda i: (i, 0)),
        out_shape=jax.ShapeDtypeStruct((N_EDGES, DIM), jnp.float32),
    )(rbf, W1, b1.reshape(1, DIM), W2, b2.reshape(1, DIM))


_NC = 2
_NS = 16
_HALF = N_NODES // _NC
_TRASH = _HALF
_ACC_ROWS = 25088
_ZERO_PER_TILE = _ACC_ROWS // _NS
_C = 48
_NB = 3
_TILE_EDGES = N_EDGES // _NS
_NCH = _TILE_EDGES // _C
_TAIL = _TILE_EDGES - _NCH * _C
assert (_NCH - 3) % 3 == 0 and _TAIL % 16 == 0 and _C % 8 == 0


def _sc_body(node_hbm, h_hbm, ef_hbm, src_hbm, dst_hbm, out_hbm,
             acc, src_v, dst_v, idx_v, g_v, h_v, ef_v,
             sem_idx, sem_row, sem_g, sem_sc):
    c = lax.axis_index("c")
    s = lax.axis_index("s")
    tile_base = s * _TILE_EDGES
    dst_lo = c * _HALF

    zeros16 = jnp.zeros((16,), jnp.float32)

    def _zero_row(j, _):
        for k in range(DIM // 16):
            g_v[0, j, pl.ds(k * 16, 16)] = zeros16
        return 0
    lax.fori_loop(0, _C, _zero_row, 0)
    zbase = s * _ZERO_PER_TILE
    for q in range(_ZERO_PER_TILE // _C):
        pltpu.sync_copy(g_v.at[0], acc.at[pl.ds(zbase + q * _C, _C)])
    if _ZERO_PER_TILE % _C:
        pltpu.sync_copy(
            g_v.at[0, pl.ds(0, _ZERO_PER_TILE % _C)],
            acc.at[pl.ds(zbase + (_ZERO_PER_TILE // _C) * _C,
                         _ZERO_PER_TILE % _C)])
    plsc.subcore_barrier()


    def start_loads(i, b, wait_sc):
        base = tile_base + i * _C
        pltpu.async_copy(src_hbm.at[pl.ds(base, _C)], src_v.at[b],
                         sem_idx[b])
        pltpu.async_copy(dst_hbm.at[pl.ds(base, _C)], dst_v.at[b],
                         sem_idx[b])
        pltpu.async_copy(h_hbm.at[pl.ds(base, _C)], h_v.at[b], sem_row[b])
        pltpu.async_copy(ef_hbm.at[pl.ds(base, _C)], ef_v.at[b], sem_row[b])

    def start_gather(i, b):
        base = tile_base + i * _C
        pltpu.make_async_copy(src_hbm.at[pl.ds(base, _C)], src_v.at[b],
                              sem_idx[b]).wait()
        pltpu.make_async_copy(dst_hbm.at[pl.ds(base, _C)], dst_v.at[b],
                              sem_idx[b]).wait()
        pltpu.async_copy(node_hbm.at[src_v.at[b]], g_v.at[b], sem_g[b])
        @pl.loop(0, _C // 16, unroll=3)
        def _fix_idx(j):
            d = dst_v[b, pl.ds(j * 16, 16)] - dst_lo
            ok = (d >= 0) & (d < _HALF)
            idx_v[b, pl.ds(j * 16, 16)] = jnp.where(ok, d, _TRASH)

    def finish(i, b, first=False):
        base = tile_base + i * _C
        pltpu.make_async_copy(h_hbm.at[pl.ds(base, _C)], h_v.at[b],
                              sem_row[b]).wait()
        pltpu.make_async_copy(ef_hbm.at[pl.ds(base, _C)], ef_v.at[b],
                              sem_row[b]).wait()
        pltpu.make_async_copy(node_hbm.at[src_v.at[b]], g_v.at[b],
                              sem_g[b]).wait()

        @pl.loop(0, _C, unroll=4)
        def _fma_row(j):
            for k in range(DIM // 16):
                sl = pl.ds(k * 16, 16)
                h_v[b, j, sl] = g_v[b, j, sl] * h_v[b, j, sl] + ef_v[b, j, sl]
        if not first:
            bp = (b - 1) % _NB
            pltpu.make_async_copy(h_v.at[bp], acc.at[idx_v.at[bp]],
                                  sem_sc[0]).wait()
        pltpu.async_copy(h_v.at[b], acc.at[idx_v.at[b]], sem_sc[0], add=True)

    start_loads(0, 0, False)
    start_loads(1, 1, False)
    start_gather(0, 0)
    start_loads(2, 2, False)
    finish(0, 0, first=True)
    start_gather(1, 1)

    @pl.loop(0, (_NCH - 3) // 3)
    def _main(g):
        for t in range(3):
            i = 1 + g * 3 + t
            start_loads(i + 2, t % 3, True)
            finish(i, (1 + t) % 3)
            start_gather(i + 1, (2 + t) % 3)

    finish(_NCH - 2, (_NCH - 2) % 3)
    start_gather(_NCH - 1, (_NCH - 1) % 3)
    finish(_NCH - 1, (_NCH - 1) % 3)
    bl = (_NCH - 1) % 3
    pltpu.make_async_copy(h_v.at[bl], acc.at[idx_v.at[bl]],
                          sem_sc[0]).wait()

    if _TAIL:
        base = tile_base + _NCH * _C
        n = _TAIL
        pltpu.sync_copy(src_hbm.at[pl.ds(base, n)], src_v.at[0, pl.ds(0, n)])
        pltpu.sync_copy(dst_hbm.at[pl.ds(base, n)], dst_v.at[0, pl.ds(0, n)])
        pltpu.sync_copy(h_hbm.at[pl.ds(base, n)], h_v.at[0, pl.ds(0, n)])
        pltpu.sync_copy(ef_hbm.at[pl.ds(base, n)], ef_v.at[0, pl.ds(0, n)])
        pltpu.sync_copy(node_hbm.at[src_v.at[0, pl.ds(0, n)]],
                        g_v.at[0, pl.ds(0, n)])

        def _fix_idx_t(j, _):
            d = dst_v[0, pl.ds(j * 16, 16)] - dst_lo
            ok = (d >= 0) & (d < _HALF)
            idx_v[0, pl.ds(j * 16, 16)] = jnp.where(ok, d, _TRASH)
            return 0
        lax.fori_loop(0, n // 16, _fix_idx_t, 0)

        def _fma_row_t(j, _):
            for k in range(DIM // 16):
                sl = pl.ds(k * 16, 16)
                h_v[0, j, sl] = g_v[0, j, sl] * h_v[0, j, sl] + ef_v[0, j, sl]
            return 0
        lax.fori_loop(0, n, _fma_row_t, 0)
        pltpu.sync_copy(h_v.at[0, pl.ds(0, n)],
                        acc.at[idx_v.at[0, pl.ds(0, n)]], add=True)

    plsc.subcore_barrier()

    dstart = jnp.minimum(s * 1568, _HALF - 1568)
    pltpu.sync_copy(acc.at[pl.ds(dstart, 1568)],
                    out_hbm.at[pl.ds(dst_lo + dstart, 1568)])


def _sc_scatter(new_node, h, edge_f, src, dst):
    mesh = plsc.VectorSubcoreMesh(core_axis_name="c", subcore_axis_name="s")
    f = pl.kernel(
        _sc_body,
        out_type=jax.ShapeDtypeStruct((N_NODES, DIM), jnp.float32),
        mesh=mesh,
        compiler_params=pltpu.CompilerParams(use_tc_tiling_on_sc=False),
        scratch_types=[
            pltpu.VMEM_SHARED((_ACC_ROWS, DIM), jnp.float32),
            pltpu.VMEM((_NB, _C), jnp.int32),
            pltpu.VMEM((_NB, _C), jnp.int32),
            pltpu.VMEM((_NB, _C), jnp.int32),
            pltpu.VMEM((_NB, _C, DIM), jnp.float32),
            pltpu.VMEM((_NB, _C, DIM), jnp.float32),
            pltpu.VMEM((_NB, _C, DIM), jnp.float32),
            [pltpu.SemaphoreType.DMA] * _NB,
            [pltpu.SemaphoreType.DMA] * _NB,
            [pltpu.SemaphoreType.DMA] * _NB,
            [pltpu.SemaphoreType.DMA] * _NB,
        ],
    )
    return f(new_node, h, edge_f, src, dst)


def kernel(new_node, rbf, edge_f, edge_index, W1, b1, W2, b2):
    src = edge_index[0].astype(jnp.int32)
    dst = edge_index[1].astype(jnp.int32)
    h = _edge_mlp(rbf, W1, b1, W2, b2)
    return _sc_scatter(new_node, h, edge_f, src, dst)


# speedup vs baseline: 1.3281x; 1.3281x over previous
import functools

import jax
import jax.numpy as jnp
from jax import lax
from jax.experimental import pallas as pl
from jax.experimental.pallas import tpu as pltpu
from jax.experimental.pallas import tpu_sc as plsc

N_NODES = 50000
N_EDGES = 800000
RBF_DIM = 128
DIM = 64
BETA = 0.5
THRESHOLD = 14.0


_MLP_BLOCK = 8000


def _mlp_body(rbf_ref, w1_ref, b1_ref, w2_ref, b2_ref, h_ref):
    x = rbf_ref[...]
    h1 = jnp.dot(x, w1_ref[...], preferred_element_type=jnp.float32) + b1_ref[...]
    bx = BETA * h1
    sp = (jnp.maximum(bx, 0.0) + jnp.log1p(jnp.exp(-jnp.abs(bx)))) / BETA
    act = jnp.where(bx > THRESHOLD, h1, sp)
    h_ref[...] = (
        jnp.dot(act, w2_ref[...], preferred_element_type=jnp.float32) + b2_ref[...]
    )


def _edge_mlp(rbf, W1, b1, W2, b2):
    nblk = N_EDGES // _MLP_BLOCK
    return pl.pallas_call(
        _mlp_body,
        grid=(nblk,),
        in_specs=[
            pl.BlockSpec((_MLP_BLOCK, RBF_DIM), lambda i: (i, 0)),
            pl.BlockSpec((RBF_DIM, DIM), lambda i: (0, 0)),
            pl.BlockSpec((1, DIM), lambda i: (0, 0)),
            pl.BlockSpec((DIM, DIM), lambda i: (0, 0)),
            pl.BlockSpec((1, DIM), lambda i: (0, 0)),
        ],
        out_specs=pl.BlockSpec((_MLP_BLOCK, DIM), lambda i: (i, 0)),
        out_shape=jax.ShapeDtypeStruct((N_EDGES, DIM), jnp.float32),
    )(rbf, W1, b1.reshape(1, DIM), W2, b2.reshape(1, DIM))


_NC = 2
_NS = 16
_HALF = N_NODES // _NC
_TRASH = _HALF
_ACC_ROWS = 25088
_ZERO_PER_TILE = _ACC_ROWS // _NS
_C = 48
_NB = 3
_TILE_EDGES = N_EDGES // _NS
_NCH = _TILE_EDGES // _C
_TAIL = _TILE_EDGES - _NCH * _C
assert (_NCH - 3) % 3 == 0 and _TAIL % 16 == 0 and _C % 8 == 0


def _sc_body(node_hbm, h_hbm, ef_hbm, src_hbm, dst_hbm, out_hbm,
             acc, src_v, dst_v, idx_v, g_v, h_v, ef_v,
             sem_idx, sem_row, sem_g, sem_sc):
    c = lax.axis_index("c")
    s = lax.axis_index("s")
    tile_base = s * _TILE_EDGES
    dst_lo = c * _HALF

    zeros16 = jnp.zeros((16,), jnp.float32)

    def _zero_row(j, _):
        for k in range(DIM // 16):
            g_v[0, j, pl.ds(k * 16, 16)] = zeros16
        return 0
    lax.fori_loop(0, _C, _zero_row, 0)
    zbase = s * _ZERO_PER_TILE
    for q in range(_ZERO_PER_TILE // _C):
        pltpu.sync_copy(g_v.at[0], acc.at[pl.ds(zbase + q * _C, _C)])
    if _ZERO_PER_TILE % _C:
        pltpu.sync_copy(
            g_v.at[0, pl.ds(0, _ZERO_PER_TILE % _C)],
            acc.at[pl.ds(zbase + (_ZERO_PER_TILE // _C) * _C,
                         _ZERO_PER_TILE % _C)])
    plsc.subcore_barrier()


    def start_loads(i, b, wait_sc):
        base = tile_base + i * _C
        pltpu.async_copy(src_hbm.at[pl.ds(base, _C)], src_v.at[b],
                         sem_idx[b])
        pltpu.async_copy(dst_hbm.at[pl.ds(base, _C)], dst_v.at[b],
                         sem_idx[b])
        pltpu.async_copy(h_hbm.at[pl.ds(base, _C)], h_v.at[b], sem_row[b])
        pltpu.async_copy(ef_hbm.at[pl.ds(base, _C)], ef_v.at[b], sem_row[b])

    def start_gather(i, b):
        base = tile_base + i * _C
        pltpu.make_async_copy(src_hbm.at[pl.ds(base, _C)], src_v.at[b],
                              sem_idx[b]).wait()
        pltpu.make_async_copy(dst_hbm.at[pl.ds(base, _C)], dst_v.at[b],
                              sem_idx[b]).wait()
        pltpu.async_copy(node_hbm.at[src_v.at[b]], g_v.at[b], sem_g[b])
        def _fix_idx(j, _):
            d = dst_v[b, pl.ds(j * 16, 16)] - dst_lo
            ok = (d >= 0) & (d < _HALF)
            idx_v[b, pl.ds(j * 16, 16)] = jnp.where(ok, d, _TRASH)
            return 0
        lax.fori_loop(0, _C // 16, _fix_idx, 0)

    def finish(i, b, first=False):
        base = tile_base + i * _C
        pltpu.make_async_copy(h_hbm.at[pl.ds(base, _C)], h_v.at[b],
                              sem_row[b]).wait()
        pltpu.make_async_copy(ef_hbm.at[pl.ds(base, _C)], ef_v.at[b],
                              sem_row[b]).wait()
        pltpu.make_async_copy(node_hbm.at[src_v.at[b]], g_v.at[b],
                              sem_g[b]).wait()

        def _fma_row(j, _):
            for k in range(DIM // 16):
                sl = pl.ds(k * 16, 16)
                h_v[b, j, sl] = g_v[b, j, sl] * h_v[b, j, sl] + ef_v[b, j, sl]
            return 0
        lax.fori_loop(0, _C, _fma_row, 0)
        if not first:
            bp = (b - 1) % _NB
            pltpu.make_async_copy(h_v.at[bp], acc.at[idx_v.at[bp]],
                                  sem_sc[0]).wait()
        pltpu.async_copy(h_v.at[b], acc.at[idx_v.at[b]], sem_sc[0], add=True)

    start_loads(0, 0, False)
    start_loads(1, 1, False)
    start_gather(0, 0)
    start_loads(2, 2, False)
    finish(0, 0, first=True)
    start_gather(1, 1)

    @pl.loop(0, (_NCH - 3) // 3)
    def _main(g):
        for t in range(3):
            i = 1 + g * 3 + t
            start_loads(i + 2, t % 3, True)
            finish(i, (1 + t) % 3)
            start_gather(i + 1, (2 + t) % 3)

    finish(_NCH - 2, (_NCH - 2) % 3)
    start_gather(_NCH - 1, (_NCH - 1) % 3)
    finish(_NCH - 1, (_NCH - 1) % 3)
    bl = (_NCH - 1) % 3
    pltpu.make_async_copy(h_v.at[bl], acc.at[idx_v.at[bl]],
                          sem_sc[0]).wait()

    if _TAIL:
        base = tile_base + _NCH * _C
        n = _TAIL
        pltpu.sync_copy(src_hbm.at[pl.ds(base, n)], src_v.at[0, pl.ds(0, n)])
        pltpu.sync_copy(dst_hbm.at[pl.ds(base, n)], dst_v.at[0, pl.ds(0, n)])
        pltpu.sync_copy(h_hbm.at[pl.ds(base, n)], h_v.at[0, pl.ds(0, n)])
        pltpu.sync_copy(ef_hbm.at[pl.ds(base, n)], ef_v.at[0, pl.ds(0, n)])
        pltpu.sync_copy(node_hbm.at[src_v.at[0, pl.ds(0, n)]],
                        g_v.at[0, pl.ds(0, n)])

        def _fix_idx_t(j, _):
            d = dst_v[0, pl.ds(j * 16, 16)] - dst_lo
            ok = (d >= 0) & (d < _HALF)
            idx_v[0, pl.ds(j * 16, 16)] = jnp.where(ok, d, _TRASH)
            return 0
        lax.fori_loop(0, n // 16, _fix_idx_t, 0)

        def _fma_row_t(j, _):
            for k in range(DIM // 16):
                sl = pl.ds(k * 16, 16)
                h_v[0, j, sl] = g_v[0, j, sl] * h_v[0, j, sl] + ef_v[0, j, sl]
            return 0
        lax.fori_loop(0, n, _fma_row_t, 0)
        pltpu.sync_copy(h_v.at[0, pl.ds(0, n)],
                        acc.at[idx_v.at[0, pl.ds(0, n)]], add=True)

    plsc.subcore_barrier()

    dstart = jnp.minimum(s * 1568, _HALF - 1568)
    pltpu.sync_copy(acc.at[pl.ds(dstart, 1568)],
                    out_hbm.at[pl.ds(dst_lo + dstart, 1568)])


def _sc_scatter(new_node, h, edge_f, src, dst):
    mesh = plsc.VectorSubcoreMesh(core_axis_name="c", subcore_axis_name="s")
    f = pl.kernel(
        _sc_body,
        out_type=jax.ShapeDtypeStruct((N_NODES, DIM), jnp.float32),
        mesh=mesh,
        compiler_params=pltpu.CompilerParams(use_tc_tiling_on_sc=False),
        scratch_types=[
            pltpu.VMEM_SHARED((_ACC_ROWS, DIM), jnp.float32),
            pltpu.VMEM((_NB, _C), jnp.int32),
            pltpu.VMEM((_NB, _C), jnp.int32),
            pltpu.VMEM((_NB, _C), jnp.int32),
            pltpu.VMEM((_NB, _C, DIM), jnp.float32),
            pltpu.VMEM((_NB, _C, DIM), jnp.float32),
            pltpu.VMEM((_NB, _C, DIM), jnp.float32),
            [pltpu.SemaphoreType.DMA] * _NB,
            [pltpu.SemaphoreType.DMA] * _NB,
            [pltpu.SemaphoreType.DMA] * _NB,
            [pltpu.SemaphoreType.DMA] * _NB,
        ],
    )
    return f(new_node, h, edge_f, src, dst)


def kernel(new_node, rbf, edge_f, edge_index, W1, b1, W2, b2):
    src = edge_index[0].astype(jnp.int32)
    dst = edge_index[1].astype(jnp.int32)
    h = _edge_mlp(rbf, W1, b1, W2, b2)
    return _sc_scatter(new_node, h, edge_f, src, dst)


# speedup vs baseline: 1.4463x; 1.0890x over previous
import functools

import jax
import jax.numpy as jnp
from jax import lax
from jax.experimental import pallas as pl
from jax.experimental.pallas import tpu as pltpu
from jax.experimental.pallas import tpu_sc as plsc

N_NODES = 50000
N_EDGES = 800000
RBF_DIM = 128
DIM = 64
BETA = 0.5
THRESHOLD = 14.0

_SPLIT = 384000


_MLP_BLOCK = 8000


def _mlp_body(rbf_ref, w1_ref, b1_ref, w2_ref, b2_ref, h_ref):
    x = rbf_ref[...]
    h1 = jnp.dot(x, w1_ref[...], preferred_element_type=jnp.float32) + b1_ref[...]
    bx = BETA * h1
    sp = (jnp.maximum(bx, 0.0) + jnp.log1p(jnp.exp(-jnp.abs(bx)))) / BETA
    act = jnp.where(bx > THRESHOLD, h1, sp)
    h_ref[...] = (
        jnp.dot(act, w2_ref[...], preferred_element_type=jnp.float32) + b2_ref[...]
    )


def _edge_mlp(rbf, W1, b1, W2, b2, blk_off, nblk):
    return pl.pallas_call(
        _mlp_body,
        grid=(nblk,),
        in_specs=[
            pl.BlockSpec((_MLP_BLOCK, RBF_DIM), lambda i: (i + blk_off, 0)),
            pl.BlockSpec((RBF_DIM, DIM), lambda i: (0, 0)),
            pl.BlockSpec((1, DIM), lambda i: (0, 0)),
            pl.BlockSpec((DIM, DIM), lambda i: (0, 0)),
            pl.BlockSpec((1, DIM), lambda i: (0, 0)),
        ],
        out_specs=pl.BlockSpec((_MLP_BLOCK, DIM), lambda i: (i, 0)),
        out_shape=jax.ShapeDtypeStruct((nblk * _MLP_BLOCK, DIM), jnp.float32),
    )(rbf, W1, b1.reshape(1, DIM), W2, b2.reshape(1, DIM))


_NC = 2
_NS = 16
_HALF = N_NODES // _NC
_TRASH = _HALF
_ACC_ROWS = 25088
_ZERO_PER_TILE = _ACC_ROWS // _NS
_C = 48
_NB = 3


def _make_sc_body(edge_base, tile_edges):
    nch = tile_edges // _C
    tail = tile_edges - nch * _C
    n_tri = (nch - 3) // 3
    n_peel = (nch - 3) - 3 * n_tri
    assert tail % 16 == 0 and tail < _C and nch >= 4

    def _sc_body(node_hbm, h_hbm, ef_hbm, src_hbm, dst_hbm, out_hbm,
                 acc, src_v, dst_v, idx_v, g_v, h_v, ef_v,
                 sem_idx, sem_row, sem_g, sem_sc):
        c = lax.axis_index("c")
        s = lax.axis_index("s")
        tile_base = edge_base + s * tile_edges
        h_tile_base = s * tile_edges
        dst_lo = c * _HALF

        zeros16 = jnp.zeros((16,), jnp.float32)

        def _zero_row(j, _):
            for k in range(DIM // 16):
                g_v[0, j, pl.ds(k * 16, 16)] = zeros16
            return 0
        lax.fori_loop(0, _C, _zero_row, 0)
        zbase = s * _ZERO_PER_TILE
        for q in range(_ZERO_PER_TILE // _C):
            pltpu.sync_copy(g_v.at[0], acc.at[pl.ds(zbase + q * _C, _C)])
        if _ZERO_PER_TILE % _C:
            pltpu.sync_copy(
                g_v.at[0, pl.ds(0, _ZERO_PER_TILE % _C)],
                acc.at[pl.ds(zbase + (_ZERO_PER_TILE // _C) * _C,
                             _ZERO_PER_TILE % _C)])
        plsc.subcore_barrier()


        def start_loads(i, b):
            base = tile_base + i * _C
            pltpu.async_copy(src_hbm.at[pl.ds(base, _C)], src_v.at[b],
                             sem_idx[b])
            pltpu.async_copy(dst_hbm.at[pl.ds(base, _C)], dst_v.at[b],
                             sem_idx[b])
            pltpu.async_copy(h_hbm.at[pl.ds(h_tile_base + i * _C, _C)],
                             h_v.at[b], sem_row[b])
            pltpu.async_copy(ef_hbm.at[pl.ds(base, _C)], ef_v.at[b],
                             sem_row[b])

        def start_gather(i, b):
            base = tile_base + i * _C
            pltpu.make_async_copy(src_hbm.at[pl.ds(base, _C)], src_v.at[b],
                                  sem_idx[b]).wait()
            pltpu.make_async_copy(dst_hbm.at[pl.ds(base, _C)], dst_v.at[b],
                                  sem_idx[b]).wait()
            pltpu.async_copy(node_hbm.at[src_v.at[b]], g_v.at[b], sem_g[b])
            def _fix_idx(j, _):
                d = dst_v[b, pl.ds(j * 16, 16)] - dst_lo
                ok = (d >= 0) & (d < _HALF)
                idx_v[b, pl.ds(j * 16, 16)] = jnp.where(ok, d, _TRASH)
                return 0
            lax.fori_loop(0, _C // 16, _fix_idx, 0)

        def finish(i, b, first=False):
            pltpu.make_async_copy(h_hbm.at[pl.ds(h_tile_base + i * _C, _C)],
                                  h_v.at[b], sem_row[b]).wait()
            pltpu.make_async_copy(ef_hbm.at[pl.ds(tile_base + i * _C, _C)],
                                  ef_v.at[b], sem_row[b]).wait()
            pltpu.make_async_copy(node_hbm.at[src_v.at[b]], g_v.at[b],
                                  sem_g[b]).wait()

            def _fma_row(j, _):
                for k in range(DIM // 16):
                    sl = pl.ds(k * 16, 16)
                    h_v[b, j, sl] = (g_v[b, j, sl] * h_v[b, j, sl]
                                     + ef_v[b, j, sl])
                return 0
            lax.fori_loop(0, _C, _fma_row, 0)
            if not first:
                bp = (b - 1) % _NB
                pltpu.make_async_copy(h_v.at[bp], acc.at[idx_v.at[bp]],
                                      sem_sc[0]).wait()
            pltpu.async_copy(h_v.at[b], acc.at[idx_v.at[b]], sem_sc[0],
                             add=True)

        start_loads(0, 0)
        start_loads(1, 1)
        start_gather(0, 0)
        start_loads(2, 2)
        finish(0, 0, first=True)
        start_gather(1, 1)

        @pl.loop(0, n_tri)
        def _main(g):
            for t in range(3):
                i = 1 + g * 3 + t
                start_loads(i + 2, t % 3)
                finish(i, (1 + t) % 3)
                start_gather(i + 1, (2 + t) % 3)

        for p in range(n_peel):
            i = 1 + 3 * n_tri + p
            start_loads(i + 2, (i + 2) % 3)
            finish(i, i % 3)
            start_gather(i + 1, (i + 1) % 3)

        finish(nch - 2, (nch - 2) % 3)
        start_gather(nch - 1, (nch - 1) % 3)
        finish(nch - 1, (nch - 1) % 3)
        bl = (nch - 1) % 3
        pltpu.make_async_copy(h_v.at[bl], acc.at[idx_v.at[bl]],
                              sem_sc[0]).wait()

        if tail:
            base = tile_base + nch * _C
            n = tail
            pltpu.sync_copy(src_hbm.at[pl.ds(base, n)],
                            src_v.at[0, pl.ds(0, n)])
            pltpu.sync_copy(dst_hbm.at[pl.ds(base, n)],
                            dst_v.at[0, pl.ds(0, n)])
            pltpu.sync_copy(h_hbm.at[pl.ds(h_tile_base + nch * _C, n)],
                            h_v.at[0, pl.ds(0, n)])
            pltpu.sync_copy(ef_hbm.at[pl.ds(base, n)],
                            ef_v.at[0, pl.ds(0, n)])
            pltpu.sync_copy(node_hbm.at[src_v.at[0, pl.ds(0, n)]],
                            g_v.at[0, pl.ds(0, n)])

            def _fix_idx_t(j, _):
                d = dst_v[0, pl.ds(j * 16, 16)] - dst_lo
                ok = (d >= 0) & (d < _HALF)
                idx_v[0, pl.ds(j * 16, 16)] = jnp.where(ok, d, _TRASH)
                return 0
            lax.fori_loop(0, n // 16, _fix_idx_t, 0)

            def _fma_row_t(j, _):
                for k in range(DIM // 16):
                    sl = pl.ds(k * 16, 16)
                    h_v[0, j, sl] = (g_v[0, j, sl] * h_v[0, j, sl]
                                     + ef_v[0, j, sl])
                return 0
            lax.fori_loop(0, n, _fma_row_t, 0)
            pltpu.sync_copy(h_v.at[0, pl.ds(0, n)],
                            acc.at[idx_v.at[0, pl.ds(0, n)]], add=True)

        plsc.subcore_barrier()

        dstart = jnp.minimum(s * 1568, _HALF - 1568)
        pltpu.sync_copy(acc.at[pl.ds(dstart, 1568)],
                        out_hbm.at[pl.ds(dst_lo + dstart, 1568)])

    return _sc_body


def _sc_scatter(body, new_node, h, edge_f, src, dst):
    mesh = plsc.VectorSubcoreMesh(core_axis_name="c", subcore_axis_name="s")
    f = pl.kernel(
        body,
        out_type=jax.ShapeDtypeStruct((N_NODES, DIM), jnp.float32),
        mesh=mesh,
        compiler_params=pltpu.CompilerParams(use_tc_tiling_on_sc=False),
        scratch_types=[
            pltpu.VMEM_SHARED((_ACC_ROWS, DIM), jnp.float32),
            pltpu.VMEM((_NB, _C), jnp.int32),
            pltpu.VMEM((_NB, _C), jnp.int32),
            pltpu.VMEM((_NB, _C), jnp.int32),
            pltpu.VMEM((_NB, _C, DIM), jnp.float32),
            pltpu.VMEM((_NB, _C, DIM), jnp.float32),
            pltpu.VMEM((_NB, _C, DIM), jnp.float32),
            [pltpu.SemaphoreType.DMA] * _NB,
            [pltpu.SemaphoreType.DMA] * _NB,
            [pltpu.SemaphoreType.DMA] * _NB,
            [pltpu.SemaphoreType.DMA] * _NB,
        ],
    )
    return f(new_node, h, edge_f, src, dst)


_SC_BODY0 = _make_sc_body(0, _SPLIT // _NS)
_SC_BODY1 = _make_sc_body(_SPLIT, (N_EDGES - _SPLIT) // _NS)


def kernel(new_node, rbf, edge_f, edge_index, W1, b1, W2, b2):
    src = edge_index[0].astype(jnp.int32)
    dst = edge_index[1].astype(jnp.int32)
    nblk0 = _SPLIT // _MLP_BLOCK
    nblk1 = (N_EDGES - _SPLIT) // _MLP_BLOCK
    h0 = _edge_mlp(rbf, W1, b1, W2, b2, 0, nblk0)
    p0 = _sc_scatter(_SC_BODY0, new_node, h0, edge_f, src, dst)
    h1 = _edge_mlp(rbf, W1, b1, W2, b2, nblk0, nblk1)
    p1 = _sc_scatter(_SC_BODY1, new_node, h1, edge_f, src, dst)
    return p0 + p1


# speedup vs baseline: 1.4674x; 1.0146x over previous
import functools

import jax
import jax.numpy as jnp
from jax import lax
from jax.experimental import pallas as pl
from jax.experimental.pallas import tpu as pltpu
from jax.experimental.pallas import tpu_sc as plsc

N_NODES = 50000
N_EDGES = 800000
RBF_DIM = 128
DIM = 64
BETA = 0.5
THRESHOLD = 14.0

_PHASES = (192000, 192000, 192000, 224000)


_MLP_BLOCK = 8000


def _mlp_body(rbf_ref, w1_ref, b1_ref, w2_ref, b2_ref, h_ref):
    x = rbf_ref[...]
    h1 = jnp.dot(x, w1_ref[...], preferred_element_type=jnp.float32) + b1_ref[...]
    bx = BETA * h1
    sp = (jnp.maximum(bx, 0.0) + jnp.log1p(jnp.exp(-jnp.abs(bx)))) / BETA
    act = jnp.where(bx > THRESHOLD, h1, sp)
    h_ref[...] = (
        jnp.dot(act, w2_ref[...], preferred_element_type=jnp.float32) + b2_ref[...]
    )


def _edge_mlp(rbf, W1, b1, W2, b2, blk_off, nblk):
    return pl.pallas_call(
        _mlp_body,
        grid=(nblk,),
        in_specs=[
            pl.BlockSpec((_MLP_BLOCK, RBF_DIM), lambda i: (i + blk_off, 0)),
            pl.BlockSpec((RBF_DIM, DIM), lambda i: (0, 0)),
            pl.BlockSpec((1, DIM), lambda i: (0, 0)),
            pl.BlockSpec((DIM, DIM), lambda i: (0, 0)),
            pl.BlockSpec((1, DIM), lambda i: (0, 0)),
        ],
        out_specs=pl.BlockSpec((_MLP_BLOCK, DIM), lambda i: (i, 0)),
        out_shape=jax.ShapeDtypeStruct((nblk * _MLP_BLOCK, DIM), jnp.float32),
    )(rbf, W1, b1.reshape(1, DIM), W2, b2.reshape(1, DIM))


_NC = 2
_NS = 16
_HALF = N_NODES // _NC
_TRASH = _HALF
_ACC_ROWS = 25088
_ZERO_PER_TILE = _ACC_ROWS // _NS
_C = 48
_NB = 3


def _make_sc_body(edge_base, tile_edges):
    nch = tile_edges // _C
    tail = tile_edges - nch * _C
    n_tri = (nch - 3) // 3
    n_peel = (nch - 3) - 3 * n_tri
    assert tail % 16 == 0 and tail < _C and nch >= 4

    def _sc_body(node_hbm, h_hbm, ef_hbm, src_hbm, dst_hbm, out_hbm,
                 acc, src_v, dst_v, idx_v, g_v, h_v, ef_v,
                 sem_idx, sem_row, sem_g, sem_sc):
        c = lax.axis_index("c")
        s = lax.axis_index("s")
        tile_base = edge_base + s * tile_edges
        h_tile_base = s * tile_edges
        dst_lo = c * _HALF

        zeros16 = jnp.zeros((16,), jnp.float32)

        def _zero_row(j, _):
            for k in range(DIM // 16):
                g_v[0, j, pl.ds(k * 16, 16)] = zeros16
            return 0
        lax.fori_loop(0, _C, _zero_row, 0)
        zbase = s * _ZERO_PER_TILE
        for q in range(_ZERO_PER_TILE // _C):
            pltpu.sync_copy(g_v.at[0], acc.at[pl.ds(zbase + q * _C, _C)])
        if _ZERO_PER_TILE % _C:
            pltpu.sync_copy(
                g_v.at[0, pl.ds(0, _ZERO_PER_TILE % _C)],
                acc.at[pl.ds(zbase + (_ZERO_PER_TILE // _C) * _C,
                             _ZERO_PER_TILE % _C)])
        plsc.subcore_barrier()


        def start_loads(i, b):
            base = tile_base + i * _C
            pltpu.async_copy(src_hbm.at[pl.ds(base, _C)], src_v.at[b],
                             sem_idx[b])
            pltpu.async_copy(dst_hbm.at[pl.ds(base, _C)], dst_v.at[b],
                             sem_idx[b])
            pltpu.async_copy(h_hbm.at[pl.ds(h_tile_base + i * _C, _C)],
                             h_v.at[b], sem_row[b])
            pltpu.async_copy(ef_hbm.at[pl.ds(base, _C)], ef_v.at[b],
                             sem_row[b])

        def start_gather(i, b):
            base = tile_base + i * _C
            pltpu.make_async_copy(src_hbm.at[pl.ds(base, _C)], src_v.at[b],
                                  sem_idx[b]).wait()
            pltpu.make_async_copy(dst_hbm.at[pl.ds(base, _C)], dst_v.at[b],
                                  sem_idx[b]).wait()
            pltpu.async_copy(node_hbm.at[src_v.at[b]], g_v.at[b], sem_g[b])
            def _fix_idx(j, _):
                d = dst_v[b, pl.ds(j * 16, 16)] - dst_lo
                ok = (d >= 0) & (d < _HALF)
                idx_v[b, pl.ds(j * 16, 16)] = jnp.where(ok, d, _TRASH)
                return 0
            lax.fori_loop(0, _C // 16, _fix_idx, 0)

        def finish(i, b, first=False):
            pltpu.make_async_copy(h_hbm.at[pl.ds(h_tile_base + i * _C, _C)],
                                  h_v.at[b], sem_row[b]).wait()
            pltpu.make_async_copy(ef_hbm.at[pl.ds(tile_base + i * _C, _C)],
                                  ef_v.at[b], sem_row[b]).wait()
            pltpu.make_async_copy(node_hbm.at[src_v.at[b]], g_v.at[b],
                                  sem_g[b]).wait()

            def _fma_row(j, _):
                for k in range(DIM // 16):
                    sl = pl.ds(k * 16, 16)
                    h_v[b, j, sl] = (g_v[b, j, sl] * h_v[b, j, sl]
                                     + ef_v[b, j, sl])
                return 0
            lax.fori_loop(0, _C, _fma_row, 0)
            if not first:
                bp = (b - 1) % _NB
                pltpu.make_async_copy(h_v.at[bp], acc.at[idx_v.at[bp]],
                                      sem_sc[0]).wait()
            pltpu.async_copy(h_v.at[b], acc.at[idx_v.at[b]], sem_sc[0],
                             add=True)

        start_loads(0, 0)
        start_loads(1, 1)
        start_gather(0, 0)
        start_loads(2, 2)
        finish(0, 0, first=True)
        start_gather(1, 1)

        @pl.loop(0, n_tri)
        def _main(g):
            for t in range(3):
                i = 1 + g * 3 + t
                start_loads(i + 2, t % 3)
                finish(i, (1 + t) % 3)
                start_gather(i + 1, (2 + t) % 3)

        for p in range(n_peel):
            i = 1 + 3 * n_tri + p
            start_loads(i + 2, (i + 2) % 3)
            finish(i, i % 3)
            start_gather(i + 1, (i + 1) % 3)

        finish(nch - 2, (nch - 2) % 3)
        start_gather(nch - 1, (nch - 1) % 3)
        finish(nch - 1, (nch - 1) % 3)
        bl = (nch - 1) % 3
        pltpu.make_async_copy(h_v.at[bl], acc.at[idx_v.at[bl]],
                              sem_sc[0]).wait()

        if tail:
            base = tile_base + nch * _C
            n = tail
            pltpu.sync_copy(src_hbm.at[pl.ds(base, n)],
                            src_v.at[0, pl.ds(0, n)])
            pltpu.sync_copy(dst_hbm.at[pl.ds(base, n)],
                            dst_v.at[0, pl.ds(0, n)])
            pltpu.sync_copy(h_hbm.at[pl.ds(h_tile_base + nch * _C, n)],
                            h_v.at[0, pl.ds(0, n)])
            pltpu.sync_copy(ef_hbm.at[pl.ds(base, n)],
                            ef_v.at[0, pl.ds(0, n)])
            pltpu.sync_copy(node_hbm.at[src_v.at[0, pl.ds(0, n)]],
                            g_v.at[0, pl.ds(0, n)])

            def _fix_idx_t(j, _):
                d = dst_v[0, pl.ds(j * 16, 16)] - dst_lo
                ok = (d >= 0) & (d < _HALF)
                idx_v[0, pl.ds(j * 16, 16)] = jnp.where(ok, d, _TRASH)
                return 0
            lax.fori_loop(0, n // 16, _fix_idx_t, 0)

            def _fma_row_t(j, _):
                for k in range(DIM // 16):
                    sl = pl.ds(k * 16, 16)
                    h_v[0, j, sl] = (g_v[0, j, sl] * h_v[0, j, sl]
                                     + ef_v[0, j, sl])
                return 0
            lax.fori_loop(0, n, _fma_row_t, 0)
            pltpu.sync_copy(h_v.at[0, pl.ds(0, n)],
                            acc.at[idx_v.at[0, pl.ds(0, n)]], add=True)

        plsc.subcore_barrier()

        dstart = jnp.minimum(s * 1568, _HALF - 1568)
        pltpu.sync_copy(acc.at[pl.ds(dstart, 1568)],
                        out_hbm.at[pl.ds(dst_lo + dstart, 1568)])

    return _sc_body


def _sc_scatter(body, new_node, h, edge_f, src, dst):
    mesh = plsc.VectorSubcoreMesh(core_axis_name="c", subcore_axis_name="s")
    f = pl.kernel(
        body,
        out_type=jax.ShapeDtypeStruct((N_NODES, DIM), jnp.float32),
        mesh=mesh,
        compiler_params=pltpu.CompilerParams(use_tc_tiling_on_sc=False),
        scratch_types=[
            pltpu.VMEM_SHARED((_ACC_ROWS, DIM), jnp.float32),
            pltpu.VMEM((_NB, _C), jnp.int32),
            pltpu.VMEM((_NB, _C), jnp.int32),
            pltpu.VMEM((_NB, _C), jnp.int32),
            pltpu.VMEM((_NB, _C, DIM), jnp.float32),
            pltpu.VMEM((_NB, _C, DIM), jnp.float32),
            pltpu.VMEM((_NB, _C, DIM), jnp.float32),
            [pltpu.SemaphoreType.DMA] * _NB,
            [pltpu.SemaphoreType.DMA] * _NB,
            [pltpu.SemaphoreType.DMA] * _NB,
            [pltpu.SemaphoreType.DMA] * _NB,
        ],
    )
    return f(new_node, h, edge_f, src, dst)


_SC_BODIES = []
_base = 0
for _n in _PHASES:
    _SC_BODIES.append(_make_sc_body(_base, _n // _NS))
    _base += _n


def kernel(new_node, rbf, edge_f, edge_index, W1, b1, W2, b2):
    src = edge_index[0].astype(jnp.int32)
    dst = edge_index[1].astype(jnp.int32)
    out = None
    base = 0
    for body, n in zip(_SC_BODIES, _PHASES):
        h = _edge_mlp(rbf, W1, b1, W2, b2, base // _MLP_BLOCK,
                      n // _MLP_BLOCK)
        p = _sc_scatter(body, new_node, h, edge_f, src, dst)
        out = p if out is None else out + p
        base += n
    return out
